# baseline (speedup 1.0000x reference)
import functools

import jax
import jax.numpy as jnp
from jax import lax
from jax.experimental import pallas as pl
from jax.experimental.pallas import tpu as pltpu
from jax.experimental.pallas import tpu_sc as plsc

NC = 2
NS = 16
NW = NC * NS

B = 16384
D_OP = 24
D_SH = 8
D = D_OP + D_SH
BPW = B // NW
CH = 128
NCH = BPW // CH
GROUP = 8


_GATHER_DNUMS = lax.GatherDimensionNumbers(
    offset_dims=(), collapsed_slice_dims=(0,), start_index_map=(0,))


def _take(v, idx):
    return lax.gather(v, idx[:, None], dimension_numbers=_GATHER_DNUMS,
                      slice_sizes=(1,),
                      mode=lax.GatherScatterMode.PROMISE_IN_BOUNDS)


def _body(node_hbm, sv_hbm, table_hbm, wb_hbm, out_hbm,
          idx_v, rows_v, comb_v, sv_v, wb_v, sem):
    wid = lax.axis_index("s") * NC + lax.axis_index("c")
    base = wid * BPW

    for j in range(NCH):
        pltpu.sync_copy(node_hbm.at[pl.ds(base + j * CH, CH)], idx_v.at[j])
    pltpu.sync_copy(sv_hbm.at[pl.ds(2 * base, 2 * BPW)], sv_v)
    pltpu.sync_copy(wb_hbm, wb_v)

    cps = [
        pltpu.async_copy(table_hbm.at[idx_v.at[j]],
                         rows_v.at[pl.ds(j * CH, CH)], sem)
        for j in range(NCH)
    ]

    iota = lax.iota(jnp.int32, 16)
    lo = iota & 7
    hi_sel = iota < 8
    wv = wb_v[pl.ds(0, 16)]
    bv = wb_v[pl.ds(16, 16)]
    w0v = _take(wv, lo)
    w1v = _take(wv, lo + 8)
    bbv = _take(bv, lo)
    shuf_hi = lo + 8

    def group_it(g, c):
        r0 = g * GROUP
        svv = sv_v[pl.ds(2 * r0, 16)]
        for k in range(GROUP):
            r = r0 + k
            v0 = rows_v[r, pl.ds(0, 16)]
            vh = rows_v[r, pl.ds(8, 16)]
            vt = _take(vh, shuf_hi)
            s0 = _take(svv, jnp.full((16,), 2 * k, jnp.int32))
            s1 = _take(svv, jnp.full((16,), 2 * k + 1, jnp.int32))
            e = s0 * w0v + s1 * w1v + bbv
            out_hi = jnp.where(hi_sel, vt, e)
            comb_v[r, pl.ds(0, 16)] = v0
            comb_v[r, pl.ds(16, 16)] = out_hi
        return c

    for j in range(NCH):
        cps[j].wait()
        lax.fori_loop(j * (CH // GROUP), (j + 1) * (CH // GROUP), group_it, 0)

    pltpu.sync_copy(comb_v, out_hbm.at[pl.ds(base, BPW), :])


@functools.lru_cache(maxsize=1)
def _sc_call():
    return pl.kernel(
        _body,
        out_type=jax.ShapeDtypeStruct((B, D), jnp.float32),
        mesh=plsc.VectorSubcoreMesh(core_axis_name="c", subcore_axis_name="s",
                                    num_cores=NC, num_subcores=NS),
        scratch_types=[
            pltpu.VMEM((NCH, CH), jnp.int32),
            pltpu.VMEM((BPW, D_OP), jnp.float32),
            pltpu.VMEM((BPW, D), jnp.float32),
            pltpu.VMEM((2 * BPW,), jnp.float32),
            pltpu.VMEM((32,), jnp.float32),
            pltpu.SemaphoreType.DMA,
        ],
        compiler_params=pltpu.CompilerParams(use_tc_tiling_on_sc=False),
    )


@jax.jit
def kernel(node_inds, shape_vals, op_table, lin_W, lin_b):
    wb = jnp.concatenate([lin_W.reshape(-1), lin_b,
                          jnp.zeros((8,), jnp.float32)])
    return _sc_call()(node_inds.astype(jnp.int32),
                      shape_vals.reshape(-1), op_table, wb)


# speedup vs baseline: 2.6653x; 2.6653x over previous
import functools

import jax
import jax.numpy as jnp
from jax import lax
from jax.experimental import pallas as pl
from jax.experimental.pallas import tpu as pltpu
from jax.experimental.pallas import tpu_sc as plsc

NC = 2
NS = 16
NW = NC * NS

B = 16384
D_OP = 24
D_SH = 8
D = D_OP + D_SH
BPW = B // NW
L = 16
NCHUNK = 4
CH = BPW // NCHUNK


_GATHER_DNUMS = lax.GatherDimensionNumbers(
    offset_dims=(), collapsed_slice_dims=(0,), start_index_map=(0,))


def _take(v, idx):
    return lax.gather(v, idx[:, None], dimension_numbers=_GATHER_DNUMS,
                      slice_sizes=(1,),
                      mode=lax.GatherScatterMode.PROMISE_IN_BOUNDS)


def _body(node_hbm, sv_hbm, table_hbm, wb_hbm, out_hbm,
          idx_v, sv_v, comb_v, wb_v, drain_v, *sems):
    wid = lax.axis_index("s") * NC + lax.axis_index("c")
    base = wid * BPW

    pltpu.sync_copy(node_hbm.at[pl.ds(base, BPW)], idx_v)
    pltpu.sync_copy(sv_hbm.at[pl.ds(2 * base, 2 * BPW)], sv_v)
    pltpu.sync_copy(wb_hbm, wb_v)

    def fire_it(sem):
        def go(i, c):
            v = idx_v[pl.ds(L * i, L)]
            for k in range(L):
                pltpu.async_copy(table_hbm.at[v[k]],
                                 comb_v.at[L * i + k, pl.ds(0, D_OP)], sem)
            return c
        return go

    for j in range(NCHUNK):
        lax.fori_loop(j * (CH // L), (j + 1) * (CH // L),
                      fire_it(sems[j]), 0)

    iota = lax.iota(jnp.int32, 16)
    lo = iota & 7
    hi_sel = iota < 8
    wv = wb_v[pl.ds(0, 16)]
    bv = wb_v[pl.ds(16, 16)]
    w0v = _take(wv, lo)
    w1v = _take(wv, lo + 8)
    bbv = _take(bv, lo)
    shuf_hi = lo + 8
    sidx = [(jnp.full((16,), 2 * k, jnp.int32),
             jnp.full((16,), 2 * k + 1, jnp.int32)) for k in range(8)]

    def group_it(g, c):
        v16 = sv_v[pl.ds(L * g, L)]
        for k in range(8):
            r = 8 * g + k
            vh = comb_v[r, pl.ds(8, 16)]
            vt = _take(vh, shuf_hi)
            s0 = _take(v16, sidx[k][0])
            s1 = _take(v16, sidx[k][1])
            e = s0 * w0v + s1 * w1v + bbv
            comb_v[r, pl.ds(16, 16)] = jnp.where(hi_sel, vt, e)
        return c

    for j in range(NCHUNK):
        pltpu.make_async_copy(
            node_hbm.at[pl.ds(0, CH * D_OP)], drain_v, sems[j]).wait()
        lax.fori_loop(j * (CH // 8), (j + 1) * (CH // 8), group_it, 0)

    pltpu.sync_copy(comb_v, out_hbm.at[pl.ds(base, BPW), :])


@functools.lru_cache(maxsize=1)
def _sc_call():
    return pl.kernel(
        _body,
        out_type=jax.ShapeDtypeStruct((B, D), jnp.float32),
        mesh=plsc.VectorSubcoreMesh(core_axis_name="c", subcore_axis_name="s",
                                    num_cores=NC, num_subcores=NS),
        scratch_types=[
            pltpu.VMEM((BPW,), jnp.int32),
            pltpu.VMEM((2 * BPW,), jnp.float32),
            pltpu.VMEM((BPW, D), jnp.float32),
            pltpu.VMEM((32,), jnp.float32),
            pltpu.VMEM((CH * D_OP,), jnp.int32),
        ] + [pltpu.SemaphoreType.DMA] * NCHUNK,
        compiler_params=pltpu.CompilerParams(use_tc_tiling_on_sc=True),
    )


@jax.jit
def kernel(node_inds, shape_vals, op_table, lin_W, lin_b):
    wb = jnp.concatenate([lin_W.reshape(-1), lin_b,
                          jnp.zeros((8,), jnp.float32)])
    return _sc_call()(node_inds.astype(jnp.int32),
                      shape_vals.reshape(-1), op_table, wb)
